# baseline (speedup 1.0000x reference)
import functools

import jax
import jax.numpy as jnp
from jax import lax
from jax.experimental import pallas as pl
from jax.experimental.pallas import tpu as pltpu
from jax.experimental.pallas import tpu_sc as plsc

_NC = 2
_NS = 16
_L = 16
_NW = _NC * _NS


@functools.partial(jax.jit, static_argnames=("n_chunks", "k_rows", "d_model"))
def _sc_gather(idx, table, *, n_chunks, k_rows, d_model):
    b_total = _NW * n_chunks * k_rows
    b_per_w = n_chunks * k_rows
    w_per_batch = idx.shape[1] // b_per_w
    max_row = table.shape[0] - 1
    mesh = plsc.VectorSubcoreMesh(core_axis_name="c", subcore_axis_name="s")

    def body(idx_hbm, tbl_hbm, out_hbm, idx_v, buf0, buf1, sem0, sem1):
        wid = lax.axis_index("s") * _NC + lax.axis_index("c")
        base = wid * b_per_w
        batch = wid // w_per_batch
        off = (wid % w_per_batch) * b_per_w

        pltpu.sync_copy(idx_hbm.at[batch, pl.ds(off, b_per_w)], idx_v)

        @pl.loop(0, n_chunks * k_rows // _L)
        def _clamp(c):
            sl = pl.ds(c * _L, _L)
            v = idx_v[sl]
            idx_v[sl] = jnp.where(v == jnp.int32(-1), jnp.int32(max_row), v)

        bufs = (buf0, buf1)
        sems = (sem0, sem1)

        def chunk_idx(cc):
            return idx_v.at[pl.ds(cc * k_rows, k_rows)]

        for b in range(2):
            pltpu.async_copy(tbl_hbm.at[chunk_idx(b)], bufs[b], sems[b])

        @pl.loop(0, n_chunks - 2, step=2)
        def _main(c):
            for b in range(2):
                cc = c + b
                pltpu.make_async_copy(tbl_hbm.at[chunk_idx(0)], bufs[b], sems[b]).wait()
                pltpu.sync_copy(bufs[b], out_hbm.at[pl.ds(base + cc * k_rows, k_rows)])
                pltpu.async_copy(tbl_hbm.at[chunk_idx(cc + 2)], bufs[b], sems[b])

        for b in range(2):
            cc = n_chunks - 2 + b
            pltpu.make_async_copy(tbl_hbm.at[chunk_idx(0)], bufs[b], sems[b]).wait()
            pltpu.sync_copy(bufs[b], out_hbm.at[pl.ds(base + cc * k_rows, k_rows)])

    run = pl.kernel(
        body,
        out_type=jax.ShapeDtypeStruct((b_total, d_model), jnp.float32),
        mesh=mesh,
        scratch_types=[
            pltpu.VMEM((n_chunks * k_rows,), jnp.int32),
            pltpu.VMEM((k_rows, d_model), jnp.float32),
            pltpu.VMEM((k_rows, d_model), jnp.float32),
            pltpu.SemaphoreType.DMA,
            pltpu.SemaphoreType.DMA,
        ],
    )
    return run(idx, table)


def kernel(indices, pos_encodings):
    d_model = pos_encodings.shape[1]
    b_total = indices.size
    k_rows = _L // 2
    n_chunks = b_total // (_NW * k_rows)
    out = _sc_gather(indices, pos_encodings, n_chunks=n_chunks, k_rows=k_rows, d_model=d_model)
    return out.reshape(indices.shape + (d_model,))


# speedup vs baseline: 1.0158x; 1.0158x over previous
import functools

import jax
import jax.numpy as jnp
from jax import lax
from jax.experimental import pallas as pl
from jax.experimental.pallas import tpu as pltpu
from jax.experimental.pallas import tpu_sc as plsc

_NC = 2
_NS = 16
_L = 16
_NW = _NC * _NS


@functools.partial(jax.jit, static_argnames=("n_chunks", "k_rows", "d_model"))
def _sc_gather(idx, table, *, n_chunks, k_rows, d_model):
    b_total = _NW * n_chunks * k_rows
    b_per_w = n_chunks * k_rows
    w_per_batch = idx.shape[1] // b_per_w
    max_row = table.shape[0] - 1
    mesh = plsc.VectorSubcoreMesh(core_axis_name="c", subcore_axis_name="s")

    def body(idx_hbm, tbl_hbm, out_hbm, idx_v, buf0, buf1, sem0, sem1):
        wid = lax.axis_index("s") * _NC + lax.axis_index("c")
        base = wid * b_per_w
        batch = wid // w_per_batch
        off = (wid % w_per_batch) * b_per_w

        pltpu.sync_copy(idx_hbm.at[batch, pl.ds(off, b_per_w)], idx_v)

        @pl.loop(0, n_chunks)
        def _clamp(c):
            sl = pl.ds(c * k_rows, _L)
            v = idx_v[sl]
            idx_v[sl] = jnp.where(v == jnp.int32(-1), jnp.int32(max_row), v)

        bufs = (buf0, buf1)
        sems = (sem0, sem1)

        def chunk_idx(cc):
            return idx_v.at[pl.ds(cc * k_rows, k_rows)]

        for b in range(2):
            pltpu.async_copy(tbl_hbm.at[chunk_idx(b)], bufs[b], sems[b])

        @pl.loop(0, n_chunks - 2, step=2)
        def _main(c):
            for b in range(2):
                cc = c + b
                pltpu.make_async_copy(tbl_hbm.at[chunk_idx(0)], bufs[b], sems[b]).wait()
                pltpu.sync_copy(bufs[b], out_hbm.at[pl.ds(base + cc * k_rows, k_rows)])
                pltpu.async_copy(tbl_hbm.at[chunk_idx(cc + 2)], bufs[b], sems[b])

        for b in range(2):
            cc = n_chunks - 2 + b
            pltpu.make_async_copy(tbl_hbm.at[chunk_idx(0)], bufs[b], sems[b]).wait()
            pltpu.sync_copy(bufs[b], out_hbm.at[pl.ds(base + cc * k_rows, k_rows)])

    run = pl.kernel(
        body,
        out_type=jax.ShapeDtypeStruct((b_total, d_model), jnp.float32),
        mesh=mesh,
        scratch_types=[
            pltpu.VMEM((n_chunks * k_rows,), jnp.int32),
            pltpu.VMEM((k_rows, d_model), jnp.float32),
            pltpu.VMEM((k_rows, d_model), jnp.float32),
            pltpu.SemaphoreType.DMA,
            pltpu.SemaphoreType.DMA,
        ],
    )
    return run(idx, table)


def kernel(indices, pos_encodings):
    d_model = pos_encodings.shape[1]
    b_total = indices.size
    k_rows = _L
    n_chunks = b_total // (_NW * k_rows)
    out = _sc_gather(indices, pos_encodings, n_chunks=n_chunks, k_rows=k_rows, d_model=d_model)
    return out.reshape(indices.shape + (d_model,))
